# Initial kernel scaffold; baseline (speedup 1.0000x reference)
import functools

import jax
import jax.numpy as jnp
from jax import lax
from jax.experimental import pallas as pl
from jax.experimental.pallas import tpu as pltpu
from jax.experimental.pallas import tpu_sc as plsc

B = 100
A = 100
DEG = 32
SLOTS = DEG + 1
D = 128
NPAIR = B * A
NROWS = NPAIR * SLOTS
CHUNK = 128
NCHUNKS = -(-NROWS // CHUNK)
LAST_R0 = NROWS - CHUNK
EPAIRS = CHUNK // SLOTS + 2

_info = plsc.get_sparse_core_info()
NW = _info.num_cores * _info.num_subcores


@functools.partial(
    pl.kernel,
    out_type=jax.ShapeDtypeStruct((NROWS, D), jnp.float32),
    mesh=plsc.VectorSubcoreMesh(core_axis_name="c", subcore_axis_name="s"),
    scratch_types=[
        pltpu.VMEM((EPAIRS, DEG), jnp.int32),
        pltpu.VMEM((CHUNK,), jnp.int32),
        pltpu.VMEM((CHUNK, D), jnp.float32),
        pltpu.SemaphoreType.DMA,
    ],
)
def _graph_gather(atoms_hbm, edges_hbm, out_hbm, e_v, idx_v, rows_v, sem):
    wid = lax.axis_index("s") * _info.num_cores + lax.axis_index("c")
    c0 = wid * NCHUNKS // NW
    c1 = (wid + 1) * NCHUNKS // NW

    def chunk_body(c, carry):
        r0 = jnp.minimum(c * CHUNK, LAST_R0)
        ebase = jnp.minimum(r0 // SLOTS, NPAIR - EPAIRS)
        pltpu.sync_copy(edges_hbm.at[pl.ds(ebase, EPAIRS), :], e_v)
        for k in range(CHUNK // 16):
            e = r0 + k * 16 + lax.iota(jnp.int32, 16)
            p = e // SLOTS
            slot = e - p * SLOTS
            ev = plsc.load_gather(e_v, [p - ebase, jnp.maximum(slot - 1, 0)])
            gidx = jnp.where(slot == 0, p, ev + (p // A) * A)
            idx_v[pl.ds(k * 16, 16)] = gidx
        pltpu.async_copy(atoms_hbm.at[idx_v], rows_v, sem).wait()
        pltpu.sync_copy(rows_v, out_hbm.at[pl.ds(r0, CHUNK), :])
        return carry

    lax.fori_loop(c0, c1, chunk_body, 0)


def kernel(atoms, edges):
    assert atoms.shape == (B, A, D) and edges.shape == (B, A, DEG)
    out = _graph_gather(atoms.reshape(NPAIR, D), edges.reshape(NPAIR, DEG))
    return out.reshape(B, A, SLOTS, D)


# speedup vs baseline: 4.3159x; 4.3159x over previous
import functools

import jax
import jax.numpy as jnp
from jax import lax
from jax.experimental import pallas as pl
from jax.experimental.pallas import tpu as pltpu
from jax.experimental.pallas import tpu_sc as plsc

B = 100
A = 100
DEG = 32
SLOTS = DEG + 1
D = 128
NPAIR = B * A
NROWS = NPAIR * SLOTS
CHUNK = 128
NCHUNKS = -(-NROWS // CHUNK)
LAST_R0 = NROWS - CHUNK
EWIN = 16

_info = plsc.get_sparse_core_info()
NW = _info.num_cores * _info.num_subcores


@functools.partial(
    pl.kernel,
    out_type=jax.ShapeDtypeStruct((NROWS, D), jnp.float32),
    mesh=plsc.VectorSubcoreMesh(core_axis_name="c", subcore_axis_name="s"),
    compiler_params=pltpu.CompilerParams(needs_layout_passes=False),
    scratch_types=[
        pltpu.VMEM((EWIN, DEG), jnp.int32),
        pltpu.VMEM((CHUNK,), jnp.int32),
        pltpu.VMEM((CHUNK, D), jnp.float32),
        pltpu.SemaphoreType.DMA,
    ],
)
def _graph_gather(atoms_hbm, edges_hbm, out_hbm, e_v, idx_v, rows_v, sem):
    wid = lax.axis_index("s") * _info.num_cores + lax.axis_index("c")
    c0 = wid * NCHUNKS // NW
    c1 = (wid + 1) * NCHUNKS // NW

    def chunk_body(c, carry):
        r0 = pl.multiple_of(jnp.minimum(c * CHUNK, LAST_R0), 8)
        ebase = pl.multiple_of(
            jnp.minimum((r0 // SLOTS) // 8 * 8, NPAIR - EWIN), 8)
        pltpu.sync_copy(edges_hbm.at[pl.ds(ebase, EWIN), :], e_v)
        for k in range(CHUNK // 16):
            e = r0 + k * 16 + lax.iota(jnp.int32, 16)
            p = lax.div(e, jnp.full((16,), SLOTS, jnp.int32))
            slot = e - p * SLOTS
            ev = plsc.load_gather(e_v, [p - ebase, jnp.maximum(slot - 1, 0)])
            base = lax.div(p, jnp.full((16,), A, jnp.int32)) * A
            idx_v[pl.ds(k * 16, 16)] = jnp.where(slot == 0, p, ev + base)
        pltpu.async_copy(atoms_hbm.at[idx_v], rows_v, sem).wait()
        pltpu.sync_copy(rows_v, out_hbm.at[pl.ds(r0, CHUNK), :])
        return carry

    lax.fori_loop(c0, c1, chunk_body, 0)


def kernel(atoms, edges):
    assert atoms.shape == (B, A, D) and edges.shape == (B, A, DEG)
    out = _graph_gather(atoms.reshape(NPAIR, D), edges.reshape(NPAIR, DEG))
    return out.reshape(B, A, SLOTS, D)


# speedup vs baseline: 4.4863x; 1.0395x over previous
import functools

import jax
import jax.numpy as jnp
from jax import lax
from jax.experimental import pallas as pl
from jax.experimental.pallas import tpu as pltpu
from jax.experimental.pallas import tpu_sc as plsc

B = 100
A = 100
DEG = 32
SLOTS = DEG + 1
D = 128
NPAIR = B * A
NROWS = NPAIR * SLOTS
CHUNK = 128
NCHUNKS = -(-NROWS // CHUNK)
LAST_R0 = NROWS - CHUNK
EWIN = 16

_info = plsc.get_sparse_core_info()
NW = _info.num_cores * _info.num_subcores


NPAD = 2 * (-(-(-(-NCHUNKS // NW)) // 2))


@functools.partial(
    pl.kernel,
    out_type=jax.ShapeDtypeStruct((NROWS, D), jnp.float32),
    mesh=plsc.VectorSubcoreMesh(core_axis_name="c", subcore_axis_name="s"),
    compiler_params=pltpu.CompilerParams(needs_layout_passes=False),
    scratch_types=[
        pltpu.VMEM((EWIN, DEG), jnp.int32),
        pltpu.VMEM((EWIN, DEG), jnp.int32),
        pltpu.VMEM((CHUNK,), jnp.int32),
        pltpu.VMEM((CHUNK,), jnp.int32),
        pltpu.VMEM((CHUNK, D), jnp.float32),
        pltpu.VMEM((CHUNK, D), jnp.float32),
        pltpu.SemaphoreType.DMA,
        pltpu.SemaphoreType.DMA,
        pltpu.SemaphoreType.DMA,
        pltpu.SemaphoreType.DMA,
    ],
)
def _graph_gather(atoms_hbm, edges_hbm, out_hbm,
                  e_v0, e_v1, idx_v0, idx_v1, rows_v0, rows_v1,
                  gsem0, gsem1, osem0, osem1):
    e_vs, idx_vs, rows_vs = (e_v0, e_v1), (idx_v0, idx_v1), (rows_v0, rows_v1)
    gsems, osems = (gsem0, gsem1), (osem0, osem1)
    wid = lax.axis_index("s") * _info.num_cores + lax.axis_index("c")
    c0 = wid * NCHUNKS // NW
    c1 = (wid + 1) * NCHUNKS // NW

    def stage_gather(c, b):
        r0 = pl.multiple_of(jnp.minimum(c * CHUNK, LAST_R0), 8)
        ebase = pl.multiple_of(
            jnp.minimum((r0 // SLOTS) // 8 * 8, NPAIR - EWIN), 8)
        pltpu.sync_copy(edges_hbm.at[pl.ds(ebase, EWIN), :], e_vs[b])
        for k in range(CHUNK // 16):
            e = r0 + k * 16 + lax.iota(jnp.int32, 16)
            p = lax.div(e, jnp.full((16,), SLOTS, jnp.int32))
            slot = e - p * SLOTS
            ev = plsc.load_gather(
                e_vs[b], [p - ebase, jnp.maximum(slot - 1, 0)])
            base = lax.div(p, jnp.full((16,), A, jnp.int32)) * A
            idx_vs[b][pl.ds(k * 16, 16)] = jnp.where(slot == 0, p, ev + base)
        cp = pltpu.async_copy(atoms_hbm.at[idx_vs[b]], rows_vs[b], gsems[b])
        return r0, cp

    def start_out(r0, b):
        pltpu.async_copy(rows_vs[b], out_hbm.at[pl.ds(r0, CHUNK), :], osems[b])

    def wait_out(b):
        pltpu.make_async_copy(
            rows_vs[b], out_hbm.at[pl.ds(0, CHUNK), :], osems[b]).wait()

    ra, cpa = stage_gather(c0, 0)
    cpa.wait()
    start_out(ra, 0)
    rb, cpb = stage_gather(c0 + 1, 1)
    cpb.wait()
    start_out(rb, 1)

    def pair_body(g, carry):
        for b in (0, 1):
            c = jnp.minimum(c0 + 2 * g + b, c1 - 1)
            wait_out(b)
            r0, cp = stage_gather(c, b)
            cp.wait()
            start_out(r0, b)
        return carry

    lax.fori_loop(1, NPAD // 2, pair_body, 0)
    wait_out(0)
    wait_out(1)


def kernel(atoms, edges):
    assert atoms.shape == (B, A, D) and edges.shape == (B, A, DEG)
    out = _graph_gather(atoms.reshape(NPAIR, D), edges.reshape(NPAIR, DEG))
    return out.reshape(B, A, SLOTS, D)


# speedup vs baseline: 6.1063x; 1.3611x over previous
import functools

import jax
import jax.numpy as jnp
from jax import lax
from jax.experimental import pallas as pl
from jax.experimental.pallas import tpu as pltpu
from jax.experimental.pallas import tpu_sc as plsc

B = 100
A = 100
DEG = 32
SLOTS = DEG + 1
D = 128
NPAIR = B * A
NROWS = NPAIR * SLOTS
CHUNK = 128
NCHUNKS = -(-NROWS // CHUNK)
LAST_R0 = NROWS - CHUNK
EWIN = 328

_info = plsc.get_sparse_core_info()
NW = _info.num_cores * _info.num_subcores


NBUF = 4
NPAD = NBUF * (-(-(-(-NCHUNKS // NW)) // NBUF))


@functools.partial(
    pl.kernel,
    out_type=jax.ShapeDtypeStruct((NROWS, D), jnp.float32),
    mesh=plsc.VectorSubcoreMesh(core_axis_name="c", subcore_axis_name="s"),
    compiler_params=pltpu.CompilerParams(needs_layout_passes=False),
    scratch_types=[
        pltpu.VMEM((EWIN, DEG), jnp.int32),
        pltpu.VMEM((NBUF, CHUNK), jnp.int32),
        pltpu.VMEM((CHUNK, D), jnp.float32),
        pltpu.VMEM((CHUNK, D), jnp.float32),
        pltpu.VMEM((CHUNK, D), jnp.float32),
        pltpu.VMEM((CHUNK, D), jnp.float32),
        pltpu.SemaphoreType.DMA,
        pltpu.SemaphoreType.DMA,
        pltpu.SemaphoreType.DMA,
        pltpu.SemaphoreType.DMA,
        pltpu.SemaphoreType.DMA,
        pltpu.SemaphoreType.DMA,
        pltpu.SemaphoreType.DMA,
        pltpu.SemaphoreType.DMA,
    ],
)
def _graph_gather(atoms_hbm, edges_hbm, out_hbm, e_v, idx_v,
                  rows_v0, rows_v1, rows_v2, rows_v3,
                  gsem0, gsem1, gsem2, gsem3, osem0, osem1, osem2, osem3):
    rows_vs = (rows_v0, rows_v1, rows_v2, rows_v3)
    gsems = (gsem0, gsem1, gsem2, gsem3)
    osems = (osem0, osem1, osem2, osem3)
    wid = lax.axis_index("s") * _info.num_cores + lax.axis_index("c")
    c0 = wid * NCHUNKS // NW
    c1 = (wid + 1) * NCHUNKS // NW

    ebase = pl.multiple_of(
        jnp.minimum((c0 * CHUNK // SLOTS) // 8 * 8, NPAIR - EWIN), 8)
    pltpu.sync_copy(edges_hbm.at[pl.ds(ebase, EWIN), :], e_v)

    def r0_of(j):
        c = jnp.minimum(c0 + j, c1 - 1)
        return pl.multiple_of(jnp.minimum(c * CHUNK, LAST_R0), 8)

    def stage_gather(j, b):
        r0 = r0_of(j)

        def idx_body(k, carry):
            e = r0 + k * 16 + lax.iota(jnp.int32, 16)
            p = lax.div(e, jnp.full((16,), SLOTS, jnp.int32))
            slot = e - p * SLOTS
            ev = plsc.load_gather(
                e_v, [p - ebase, jnp.maximum(slot - 1, 0)])
            base = lax.div(p, jnp.full((16,), A, jnp.int32)) * A
            idx_v[b, pl.ds(k * 16, 16)] = jnp.where(slot == 0, p, ev + base)
            return carry

        lax.fori_loop(0, CHUNK // 16, idx_body, 0)
        pltpu.async_copy(atoms_hbm.at[idx_v.at[b]], rows_vs[b], gsems[b])

    def wait_gather(b):
        pltpu.make_async_copy(
            atoms_hbm.at[idx_v.at[b]], rows_vs[b], gsems[b]).wait()

    def start_out(j, b):
        pltpu.async_copy(
            rows_vs[b], out_hbm.at[pl.ds(r0_of(j), CHUNK), :], osems[b])

    def wait_out(b):
        pltpu.make_async_copy(
            rows_vs[b], out_hbm.at[pl.ds(0, CHUNK), :], osems[b]).wait()

    stage_gather(0, 0)
    stage_gather(1, 1)
    stage_gather(2, 2)
    wait_gather(0)
    start_out(0, 0)
    stage_gather(3, 3)
    wait_gather(1)
    start_out(1, 1)

    def quad_body(g, carry):
        for b in range(NBUF):
            j = NBUF * g + b
            wait_out(b)
            stage_gather(j, b)
            b2 = (b + 2) % NBUF
            wait_gather(b2)
            start_out(j - 2, b2)
        return carry

    lax.fori_loop(1, NPAD // NBUF, quad_body, 0)
    wait_gather(2)
    start_out(NPAD - 2, 2)
    wait_gather(3)
    start_out(NPAD - 1, 3)
    for b in range(NBUF):
        wait_out(b)


def kernel(atoms, edges):
    assert atoms.shape == (B, A, D) and edges.shape == (B, A, DEG)
    out = _graph_gather(atoms.reshape(NPAIR, D), edges.reshape(NPAIR, DEG))
    return out.reshape(B, A, SLOTS, D)


# speedup vs baseline: 19.7399x; 3.2327x over previous
import functools

import jax
import jax.numpy as jnp
from jax import lax
from jax.experimental import pallas as pl
from jax.experimental.pallas import tpu as pltpu
from jax.experimental.pallas import tpu_sc as plsc

B = 100
A = 100
DEG = 32
SLOTS = DEG + 1
D = 128
NPAIR = B * A
NUNITS = B * SLOTS
GROWS = 112
EWIN = 160

_info = plsc.get_sparse_core_info()
NW = _info.num_cores * _info.num_subcores

NBUF = 4
NPAD = NBUF * (-(-(-(-NUNITS // NW)) // NBUF))


@functools.partial(
    pl.kernel,
    out_type=jax.ShapeDtypeStruct((B, SLOTS, A, D), jnp.float32),
    mesh=plsc.VectorSubcoreMesh(core_axis_name="c", subcore_axis_name="s"),
    compiler_params=pltpu.CompilerParams(needs_layout_passes=False),
    scratch_types=[
        pltpu.VMEM((EWIN, A), jnp.int32),
        pltpu.VMEM((NBUF, GROWS), jnp.int32),
        pltpu.VMEM((GROWS, D), jnp.float32),
        pltpu.VMEM((GROWS, D), jnp.float32),
        pltpu.VMEM((GROWS, D), jnp.float32),
        pltpu.VMEM((GROWS, D), jnp.float32),
        pltpu.SemaphoreType.DMA,
        pltpu.SemaphoreType.DMA,
        pltpu.SemaphoreType.DMA,
        pltpu.SemaphoreType.DMA,
        pltpu.SemaphoreType.DMA,
        pltpu.SemaphoreType.DMA,
        pltpu.SemaphoreType.DMA,
        pltpu.SemaphoreType.DMA,
    ],
)
def _graph_gather(atoms_hbm, edges_hbm, out_hbm, e_v, idx_v,
                  rows_v0, rows_v1, rows_v2, rows_v3,
                  gsem0, gsem1, gsem2, gsem3, osem0, osem1, osem2, osem3):
    rows_vs = (rows_v0, rows_v1, rows_v2, rows_v3)
    gsems = (gsem0, gsem1, gsem2, gsem3)
    osems = (osem0, osem1, osem2, osem3)
    wid = lax.axis_index("s") * _info.num_cores + lax.axis_index("c")
    c0 = wid * NUNITS // NW
    c1 = (wid + 1) * NUNITS // NW

    ebase = pl.multiple_of(
        jnp.minimum((c0 // SLOTS) * DEG, B * DEG - EWIN), 8)
    pltpu.sync_copy(edges_hbm.at[pl.ds(ebase, EWIN), :], e_v)

    def unit_of(j):
        return jnp.minimum(c0 + j, c1 - 1)

    def stage_gather(j, b):
        u = unit_of(j)
        ub = u // SLOTS
        slot = u - ub * SLOTS
        row0 = ub * A
        erow = jnp.maximum(ub * DEG + slot - 1 - ebase, 0)

        @pl.when(slot == 0)
        def _self():
            def body(k, carry):
                a = jnp.minimum(k * 16 + lax.iota(jnp.int32, 16), A - 1)
                idx_v[b, pl.ds(k * 16, 16)] = row0 + a
                return carry
            lax.fori_loop(0, GROWS // 16, body, 0)

        @pl.when(slot != 0)
        def _neigh():
            erow16 = jnp.full((16,), 0, jnp.int32) + erow
            def body(k, carry):
                a = jnp.minimum(k * 16 + lax.iota(jnp.int32, 16), A - 1)
                ev = plsc.load_gather(e_v, [erow16, a])
                idx_v[b, pl.ds(k * 16, 16)] = row0 + ev
                return carry
            lax.fori_loop(0, GROWS // 16, body, 0)

        pltpu.async_copy(atoms_hbm.at[idx_v.at[b]], rows_vs[b], gsems[b])

    def wait_gather(b):
        pltpu.make_async_copy(
            atoms_hbm.at[idx_v.at[b]], rows_vs[b], gsems[b]).wait()

    def start_out(j, b):
        u = unit_of(j)
        ub = u // SLOTS
        slot = u - ub * SLOTS
        pltpu.async_copy(
            rows_vs[b].at[pl.ds(0, A)], out_hbm.at[ub, slot], osems[b])

    def wait_out(b):
        pltpu.make_async_copy(
            rows_vs[b].at[pl.ds(0, A)], out_hbm.at[0, 0], osems[b]).wait()

    stage_gather(0, 0)
    stage_gather(1, 1)
    stage_gather(2, 2)
    wait_gather(0)
    start_out(0, 0)
    stage_gather(3, 3)
    wait_gather(1)
    start_out(1, 1)

    def quad_body(g, carry):
        for b in range(NBUF):
            j = NBUF * g + b
            wait_out(b)
            stage_gather(j, b)
            b2 = (b + 2) % NBUF
            wait_gather(b2)
            start_out(j - 2, b2)
        return carry

    lax.fori_loop(1, NPAD // NBUF, quad_body, 0)
    wait_gather(2)
    start_out(NPAD - 2, 2)
    wait_gather(3)
    start_out(NPAD - 1, 3)
    for b in range(NBUF):
        wait_out(b)


def kernel(atoms, edges):
    assert atoms.shape == (B, A, D) and edges.shape == (B, A, DEG)
    ef = edges.transpose(0, 2, 1).reshape(B * DEG, A)
    out4 = _graph_gather(atoms.reshape(NPAIR, D), ef)
    return out4.transpose(0, 2, 1, 3)


# speedup vs baseline: 23.3774x; 1.1843x over previous
import functools

import jax
import jax.numpy as jnp
from jax import lax
from jax.experimental import pallas as pl
from jax.experimental.pallas import tpu as pltpu
from jax.experimental.pallas import tpu_sc as plsc

B = 100
A = 100
DEG = 32
SLOTS = DEG + 1
D = 128
NPAIR = B * A
NUNITS = B * SLOTS
IDXW = 112
EWIN = 160

_info = plsc.get_sparse_core_info()
NW = _info.num_cores * _info.num_subcores

NBUF = 4
NPAD = NBUF * (-(-(-(-NUNITS // NW)) // NBUF))


@functools.partial(
    pl.kernel,
    out_type=jax.ShapeDtypeStruct((B, SLOTS, A, D), jnp.float32),
    mesh=plsc.VectorSubcoreMesh(core_axis_name="c", subcore_axis_name="s"),
    compiler_params=pltpu.CompilerParams(needs_layout_passes=False),
    scratch_types=[
        pltpu.VMEM((EWIN, A), jnp.int32),
        pltpu.VMEM((NBUF, IDXW), jnp.int32),
        pltpu.VMEM((A, D), jnp.float32),
        pltpu.VMEM((A, D), jnp.float32),
        pltpu.VMEM((A, D), jnp.float32),
        pltpu.VMEM((A, D), jnp.float32),
        pltpu.SemaphoreType.DMA,
        pltpu.SemaphoreType.DMA,
        pltpu.SemaphoreType.DMA,
        pltpu.SemaphoreType.DMA,
        pltpu.SemaphoreType.DMA,
        pltpu.SemaphoreType.DMA,
        pltpu.SemaphoreType.DMA,
        pltpu.SemaphoreType.DMA,
    ],
)
def _graph_gather(atoms_hbm, edges_hbm, out_hbm, e_v, idx_v,
                  rows_v0, rows_v1, rows_v2, rows_v3,
                  gsem0, gsem1, gsem2, gsem3, osem0, osem1, osem2, osem3):
    rows_vs = (rows_v0, rows_v1, rows_v2, rows_v3)
    gsems = (gsem0, gsem1, gsem2, gsem3)
    osems = (osem0, osem1, osem2, osem3)
    wid = lax.axis_index("s") * _info.num_cores + lax.axis_index("c")
    c0 = wid * NUNITS // NW
    c1 = (wid + 1) * NUNITS // NW

    ebase = pl.multiple_of(
        jnp.minimum((c0 // SLOTS) * DEG, B * DEG - EWIN), 8)
    pltpu.sync_copy(edges_hbm.at[pl.ds(ebase, EWIN), :], e_v)

    def unit_of(j):
        return jnp.minimum(c0 + j, c1 - 1)

    def stage_gather(j, b):
        u = unit_of(j)
        ub = u // SLOTS
        slot = u - ub * SLOTS
        row0 = ub * A
        erow = jnp.maximum(ub * DEG + slot - 1 - ebase, 0)

        @pl.when(slot == 0)
        def _self():
            def body(k, carry):
                a = jnp.minimum(k * 16 + lax.iota(jnp.int32, 16), A - 1)
                idx_v[b, pl.ds(k * 16, 16)] = row0 + a
                return carry
            lax.fori_loop(0, IDXW // 16, body, 0)

        @pl.when(slot != 0)
        def _neigh():
            erow16 = jnp.full((16,), 0, jnp.int32) + erow
            def body(k, carry):
                a = jnp.minimum(k * 16 + lax.iota(jnp.int32, 16), A - 1)
                ev = plsc.load_gather(e_v, [erow16, a])
                idx_v[b, pl.ds(k * 16, 16)] = row0 + ev
                return carry
            lax.fori_loop(0, IDXW // 16, body, 0)

        pltpu.async_copy(
            atoms_hbm.at[idx_v.at[b, pl.ds(0, A)]], rows_vs[b], gsems[b])

    def wait_gather(b):
        pltpu.make_async_copy(
            atoms_hbm.at[idx_v.at[b, pl.ds(0, A)]], rows_vs[b], gsems[b]).wait()

    def start_out(j, b):
        u = unit_of(j)
        ub = u // SLOTS
        slot = u - ub * SLOTS
        pltpu.async_copy(rows_vs[b], out_hbm.at[ub, slot], osems[b])

    def wait_out(b):
        pltpu.make_async_copy(rows_vs[b], out_hbm.at[0, 0], osems[b]).wait()

    stage_gather(0, 0)
    stage_gather(1, 1)
    stage_gather(2, 2)
    wait_gather(0)
    start_out(0, 0)
    stage_gather(3, 3)
    wait_gather(1)
    start_out(1, 1)

    def quad_body(g, carry):
        for b in range(NBUF):
            j = NBUF * g + b
            wait_out(b)
            stage_gather(j, b)
            b2 = (b + 2) % NBUF
            wait_gather(b2)
            start_out(j - 2, b2)
        return carry

    lax.fori_loop(1, NPAD // NBUF, quad_body, 0)
    wait_gather(2)
    start_out(NPAD - 2, 2)
    wait_gather(3)
    start_out(NPAD - 1, 3)
    for b in range(NBUF):
        wait_out(b)


def kernel(atoms, edges):
    assert atoms.shape == (B, A, D) and edges.shape == (B, A, DEG)
    ef = edges.transpose(0, 2, 1).reshape(B * DEG, A)
    out4 = _graph_gather(atoms.reshape(NPAIR, D), ef)
    return out4.transpose(0, 2, 1, 3)
